# Initial kernel scaffold; baseline (speedup 1.0000x reference)
import jax
import jax.numpy as jnp
from jax.experimental import pallas as pl


def kernel(node_feat, edge_feat, edge_index, in_w, out_w, src_w, dst_w, nloop_w, eloop_w, nbias, ebias, nw1, nb1, nw2, nb2, ew1, eb1, ew2, eb2):
    raise NotImplementedError("write your pallas kernel here")


# speedup vs baseline: 4.0546x; 4.0546x over previous
import functools

import jax
import jax.numpy as jnp
from jax import lax
from jax.experimental import pallas as pl
from jax.experimental.pallas import tpu as pltpu
from jax.experimental.pallas import tpu_sc as plsc

N = 10000
E = 320000
D = 128
H = 128

NC = 2
NS = 16
NW = NC * NS
CHUNK = 128
NCHUNKS = E // CHUNK
KMAX = -(-NCHUNKS // NW)
DEGW = 16
NP = 10240
ZROWS = NP // NS
L = 16

_MESH = plsc.VectorSubcoreMesh(core_axis_name="c", subcore_axis_name="s")


@functools.partial(
    pl.kernel,
    out_type=(
        jax.ShapeDtypeStruct((NC, NP, D), jnp.float32),
        jax.ShapeDtypeStruct((NC * NP,), jnp.float32),
    ),
    mesh=_MESH,
    scratch_types=[
        pltpu.VMEM((CHUNK, D), jnp.float32),
        pltpu.VMEM((CHUNK,), jnp.int32),
        pltpu.VMEM((CHUNK,), jnp.int32),
        pltpu.VMEM((CHUNK,), jnp.float32),
        pltpu.VMEM_SHARED((NP, D), jnp.float32),
        pltpu.VMEM_SHARED((NP,), jnp.float32),
    ],
)
def _sc_scatter(ef_hbm, dst_hbm, src_hbm, zrows_hbm, zdeg_hbm, ones_hbm,
                s_out, deg_out, rows_v, dsti_v, srci_v, ones_v, s_acc, d_acc):
    c = lax.axis_index("c")
    s = lax.axis_index("s")
    wid = s * NC + c
    pltpu.sync_copy(zrows_hbm, s_acc.at[pl.ds(s * ZROWS, ZROWS)])
    pltpu.sync_copy(zdeg_hbm, d_acc.at[pl.ds(s * ZROWS, ZROWS)])
    pltpu.sync_copy(ones_hbm, ones_v)
    plsc.subcore_barrier()

    def body(k, carry):
        chunk = wid + k * NW

        @pl.when(chunk < NCHUNKS)
        def _():
            base = chunk * CHUNK
            pltpu.sync_copy(dst_hbm.at[pl.ds(base, CHUNK)], dsti_v)
            pltpu.sync_copy(src_hbm.at[pl.ds(base, CHUNK)], srci_v)
            pltpu.sync_copy(ef_hbm.at[pl.ds(base, CHUNK)], rows_v)
            pltpu.sync_copy(rows_v, s_acc.at[dsti_v], add=True)
            pltpu.sync_copy(ones_v, d_acc.at[srci_v], add=True)

        return carry

    lax.fori_loop(0, KMAX, body, 0)
    plsc.subcore_barrier()
    pltpu.sync_copy(s_acc.at[pl.ds(s * ZROWS, ZROWS)], s_out.at[c, pl.ds(s * ZROWS, ZROWS)])
    pltpu.sync_copy(d_acc.at[pl.ds(s * ZROWS, ZROWS)], deg_out.at[pl.ds(c * NP + s * ZROWS, ZROWS)])


@functools.partial(
    pl.kernel,
    out_type=(
        jax.ShapeDtypeStruct((E, H), jnp.float32),
        jax.ShapeDtypeStruct((E, H), jnp.float32),
        jax.ShapeDtypeStruct((E,), jnp.float32),
    ),
    mesh=_MESH,
    scratch_types=[
        pltpu.VMEM((CHUNK,), jnp.int32),
        pltpu.VMEM((CHUNK,), jnp.int32),
        pltpu.VMEM((CHUNK, H), jnp.float32),
        pltpu.VMEM((CHUNK, H), jnp.float32),
        pltpu.VMEM((CHUNK,), jnp.float32),
        pltpu.VMEM((NP,), jnp.float32),
        pltpu.VMEM((NP,), jnp.float32),
        pltpu.SemaphoreType.DMA,
        pltpu.SemaphoreType.DMA,
    ],
    compiler_params=pltpu.CompilerParams(needs_layout_passes=False),
)
def _sc_gather(adst_hbm, asrc_hbm, deg_hbm, dst_hbm, src_hbm,
               msgd_out, msgs_out, degv_out,
               dsti_v, srci_v, buf_a, buf_b, buf_f, d0_v, d1_v, sem_a, sem_b):
    c = lax.axis_index("c")
    s = lax.axis_index("s")
    wid = s * NC + c
    pltpu.sync_copy(deg_hbm.at[pl.ds(0, NP)], d0_v)
    pltpu.sync_copy(deg_hbm.at[pl.ds(NP, NP)], d1_v)

    def body(k, carry):
        chunk = wid + k * NW

        @pl.when(chunk < NCHUNKS)
        def _():
            base = chunk * CHUNK
            pltpu.sync_copy(dst_hbm.at[pl.ds(base, CHUNK)], dsti_v)
            pltpu.sync_copy(src_hbm.at[pl.ds(base, CHUNK)], srci_v)
            cp_a = pltpu.async_copy(adst_hbm.at[dsti_v], buf_a, sem_a)
            cp_b = pltpu.async_copy(asrc_hbm.at[srci_v], buf_b, sem_b)
            for j in range(CHUNK // L):
                idx = dsti_v[pl.ds(j * L, L)]
                buf_f[pl.ds(j * L, L)] = (
                    plsc.load_gather(d0_v, [idx]) + plsc.load_gather(d1_v, [idx])
                )
            cp_a.wait()
            pltpu.sync_copy(buf_a, msgd_out.at[pl.ds(base, CHUNK)])
            cp_b.wait()
            pltpu.sync_copy(buf_b, msgs_out.at[pl.ds(base, CHUNK)])
            pltpu.sync_copy(buf_f, degv_out.at[pl.ds(base, CHUNK)])

        return carry

    lax.fori_loop(0, KMAX, body, 0)


BN = 1000
BE = 640


def _tc_tables_body(nf, dstw, srcw, adst_o, asrc_o):
    x = nf[...]
    adst_o[...] = x @ dstw[...]
    asrc_o[...] = x @ srcw[...]


def _tc_node_body(nf, sp, nloopw, inw, nb, nw1t, nb1, nw2t, nb2, n_o):
    x = nf[...]
    ssum = sp[0] + sp[1]
    npre = x @ nloopw[...] - ssum @ inw[...] + nb[...]
    h = jnp.maximum(npre @ nw1t[...] + nb1[...], 0.0)
    n_o[...] = h @ nw2t[...] + nb2[...]


_LOG2E = 1.4426950408889634


def _tc_edge_body(ef, msgd, msgs, dv, w2, eb, ew1t, eb1, ew2t, eb2, e_o):
    p = ef[...] @ w2[...]
    d = dv[0]
    eye = (lax.broadcasted_iota(jnp.int32, (128, 128), 0)
           == lax.broadcasted_iota(jnp.int32, (128, 128), 1)).astype(jnp.float32)
    cols = [
        jnp.sum(jnp.broadcast_to(d[r:r + 1, :], (128, 128)) * eye, axis=1,
                keepdims=True)
        for r in range(BE // 128)
    ]
    dcol = jnp.concatenate(cols, axis=0)
    f = 2.0 * (1.0 + _LOG2E * jnp.log(1.0 + dcol))
    epre = p[:, :H] + f * p[:, H:] + msgd[...] - msgs[...] + eb[...]
    h = jnp.maximum(epre @ ew1t[...] + eb1[...], 0.0)
    e_o[...] = h @ ew2t[...] + eb2[...]


def _row_spec(rows, cols):
    return pl.BlockSpec((rows, cols), lambda i: (i, 0))


def _full_spec(shape):
    nd = len(shape)
    return pl.BlockSpec(shape, lambda i: (0,) * nd)


def kernel(node_feat, edge_feat, edge_index, in_w, out_w, src_w, dst_w, nloop_w,
           eloop_w, nbias, ebias, nw1, nb1, nw2, nb2, ew1, eb1, ew2, eb2):
    src = edge_index[0]
    dst = edge_index[1]
    zrows = jnp.zeros((ZROWS, D), jnp.float32)
    zdeg = jnp.zeros((ZROWS,), jnp.float32)
    ones = jnp.ones((CHUNK,), jnp.float32)

    spart, deg1d = _sc_scatter(edge_feat, dst, src, zrows, zdeg, ones)

    adst, asrc = pl.pallas_call(
        _tc_tables_body,
        grid=(N // BN,),
        in_specs=[
            _row_spec(BN, D),
            _full_spec((D, H)),
            _full_spec((D, H)),
        ],
        out_specs=[_row_spec(BN, H), _row_spec(BN, H)],
        out_shape=[
            jax.ShapeDtypeStruct((N, H), jnp.float32),
            jax.ShapeDtypeStruct((N, H), jnp.float32),
        ],
    )(node_feat, dst_w, src_w)

    msgd, msgs, degv = _sc_gather(adst, asrc, deg1d, dst, src)

    nb2d = nbias.reshape(1, H)
    eb2d = ebias.reshape(1, H)
    nb1_2d = nb1.reshape(1, H)
    nb2_2d = nb2.reshape(1, H)
    eb1_2d = eb1.reshape(1, H)
    eb2_2d = eb2.reshape(1, H)

    n = pl.pallas_call(
        _tc_node_body,
        grid=(N // BN,),
        in_specs=[
            _row_spec(BN, D),
            pl.BlockSpec((NC, BN, D), lambda i: (0, i, 0)),
            _full_spec((D, H)),
            _full_spec((D, H)),
            _full_spec((1, H)),
            _full_spec((H, H)),
            _full_spec((1, H)),
            _full_spec((H, H)),
            _full_spec((1, H)),
        ],
        out_specs=_row_spec(BN, H),
        out_shape=jax.ShapeDtypeStruct((N, H), jnp.float32),
    )(node_feat, spart, nloop_w, in_w, nb2d, nw1.T, nb1_2d, nw2.T, nb2_2d)

    w2 = jnp.concatenate([eloop_w, src_w - dst_w], axis=1)
    degv3d = degv.reshape(E // BE, BE // 128, 128)
    e = pl.pallas_call(
        _tc_edge_body,
        grid=(E // BE,),
        in_specs=[
            _row_spec(BE, D),
            _row_spec(BE, H),
            _row_spec(BE, H),
            pl.BlockSpec((1, BE // 128, 128), lambda i: (i, 0, 0)),
            _full_spec((D, 2 * H)),
            _full_spec((1, H)),
            _full_spec((H, H)),
            _full_spec((1, H)),
            _full_spec((H, H)),
            _full_spec((1, H)),
        ],
        out_specs=_row_spec(BE, H),
        out_shape=jax.ShapeDtypeStruct((E, H), jnp.float32),
    )(edge_feat, msgd, msgs, degv3d, w2, eb2d, ew1.T, eb1_2d, ew2.T, eb2_2d)

    return (n, e)
